# Initial kernel scaffold; baseline (speedup 1.0000x reference)
import jax
import jax.numpy as jnp
from jax.experimental import pallas as pl


def kernel(encoded_sents, mask, keys, U, V, W):
    raise NotImplementedError("write your pallas kernel here")


# speedup vs baseline: 2.0811x; 2.0811x over previous
import jax
import jax.numpy as jnp
from jax.experimental import pallas as pl
from jax.experimental.pallas import tpu as pltpu

B, S, K, D = 1024, 50, 20, 128
BB = 128


def _entity_scan_kernel(x_ref, m_ref, keys_ref, u_ref, v_ref, w_ref, out_ref,
                        xw_s, xk_s):
    keys = keys_ref[...]
    u = u_ref[...]
    v = v_ref[...]
    w = w_ref[...]

    kv = jnp.dot(keys.reshape(BB * K, D), v,
                 preferred_element_type=jnp.float32).reshape(BB, K, D)
    x_all = x_ref[...]
    xw_s[...] = jnp.dot(x_all.reshape(BB * S, D), w,
                        preferred_element_type=jnp.float32).reshape(BB, S, D)
    xk = jax.lax.dot_general(
        x_all, keys,
        dimension_numbers=(((2,), (2,)), ((0,), (0,))),
        preferred_element_type=jnp.float32)
    m = m_ref[...]
    xk_s[...] = xk + (m[..., None] - 1.0) * 1e30

    def step(t, h):
        x_t = x_ref[:, pl.ds(t, 1), :]
        g = jax.nn.sigmoid(jnp.sum(h * x_t, axis=2) + xk_s[:, t, :])
        hu = jnp.dot(h.reshape(BB * K, D), u,
                     preferred_element_type=jnp.float32).reshape(BB, K, D)
        h_tilda = jax.nn.relu(hu + kv + xw_s[:, pl.ds(t, 1), :])
        upd = h + g[..., None] * h_tilda
        denom = jnp.sqrt(jnp.maximum(jnp.sum(upd * upd, axis=2, keepdims=True),
                                     1e-12))
        return upd / denom

    h0 = jnp.zeros((BB, K, D), dtype=jnp.float32)
    out_ref[...] = jax.lax.fori_loop(0, S, step, h0)


@jax.jit
def kernel(encoded_sents, mask, keys, U, V, W):
    nb = B // BB
    mask_f = mask.astype(jnp.float32)
    return pl.pallas_call(
        _entity_scan_kernel,
        grid=(nb,),
        in_specs=[
            pl.BlockSpec((BB, S, D), lambda b: (b, 0, 0)),
            pl.BlockSpec((BB, S), lambda b: (b, 0)),
            pl.BlockSpec((BB, K, D), lambda b: (b, 0, 0)),
            pl.BlockSpec((D, D), lambda b: (0, 0)),
            pl.BlockSpec((D, D), lambda b: (0, 0)),
            pl.BlockSpec((D, D), lambda b: (0, 0)),
        ],
        out_specs=pl.BlockSpec((BB, K, D), lambda b: (b, 0, 0)),
        out_shape=jax.ShapeDtypeStruct((B, K, D), jnp.float32),
        scratch_shapes=[
            pltpu.VMEM((BB, S, D), jnp.float32),
            pltpu.VMEM((BB, S, K), jnp.float32),
        ],
    )(encoded_sents, mask_f, keys, U, V, W)
